# baseline (speedup 1.0000x reference)
import functools

import jax
import jax.numpy as jnp
from jax import lax
from jax.experimental import pallas as pl
from jax.experimental.pallas import tpu as pltpu
from jax.experimental.pallas import tpu_sc as plsc

_BATCH = 4096
_SEQ = 200
_V = 1000000
_D = 32
_NC, _NS = 2, 16
_NW = _NC * _NS
_CB = 512
_BQ = _BATCH // _CB
_BBL = _CB // 128
_UNITS = _SEQ * _BQ
_UPW = _UNITS // _NW


def _build():
    mesh = plsc.VectorSubcoreMesh(core_axis_name="c", subcore_axis_name="s")

    @functools.partial(
        pl.kernel,
        mesh=mesh,
        out_type=jax.ShapeDtypeStruct((_SEQ, _D // 8, _BATCH // 128, 8, 128),
                                      jnp.float32),
        compiler_params=pltpu.CompilerParams(use_tc_tiling_on_sc=False,
                                             needs_layout_passes=False),
        scratch_types=[
            pltpu.VMEM((_CB,), jnp.int32),
            pltpu.VMEM((_CB,), jnp.int32),
            pltpu.VMEM((_CB, _D), jnp.float32),
            pltpu.VMEM((_CB, _D), jnp.float32),
            pltpu.VMEM((_D // 8, _BBL, 8, 128), jnp.float32),
            pltpu.VMEM((_D // 8, _BBL, 8, 128), jnp.float32),
            pltpu.SemaphoreType.DMA,
            pltpu.SemaphoreType.DMA,
            pltpu.SemaphoreType.DMA,
            pltpu.SemaphoreType.DMA,
            pltpu.SemaphoreType.DMA,
            pltpu.SemaphoreType.DMA,
        ],
    )
    def gather_kernel(idx_hbm, table_hbm, z_hbm, ib0, ib1, gb0, gb1, sb0, sb1,
                      is0, is1, gs0, gs1, ws0, ws1):
        wid = lax.axis_index("s") * _NC + lax.axis_index("c")
        u0 = wid * _UPW
        iota = lax.iota(jnp.int32, 16)

        ibufs = (ib0, ib1)
        gbufs = (gb0, gb1)
        sbufs = (sb0, sb1)
        isems = (is0, is1)
        gsems = (gs0, gs1)
        wsems = (ws0, ws1)

        def idx_src(u):
            l = u // _BQ
            bq = u % _BQ
            return idx_hbm.at[pl.ds(l * _BATCH + bq * _CB, _CB)]

        def start_idx(u, p):
            pltpu.async_copy(idx_src(u), ibufs[p], isems[p])

        def wait_idx(u, p):
            pltpu.make_async_copy(idx_src(u), ibufs[p], isems[p]).wait()

        def start_gather(p):
            pltpu.async_copy(table_hbm.at[ibufs[p]], gbufs[p], gsems[p])

        def wait_gather(p):
            pltpu.make_async_copy(table_hbm.at[ibufs[p]], gbufs[p],
                                  gsems[p]).wait()

        def wb_pairs(u, p):
            l = u // _BQ
            bq = u % _BQ
            return [(sbufs[p].at[jj],
                     z_hbm.at[l, jj, pl.ds(bq * _BBL, _BBL)])
                    for jj in range(_D // 8)]

        def start_wb(u, p):
            for src, dst in wb_pairs(u, p):
                pltpu.async_copy(src, dst, wsems[p])

        def wait_wb(u, p):
            for src, dst in wb_pairs(u, p):
                pltpu.make_async_copy(src, dst, wsems[p]).wait()

        def shuffle(p):
            gb = gbufs[p]
            sb = sbufs[p]

            def body(t, carry):
                jj = t // _BBL
                bbl = t % _BBL
                rows = [bbl * 128 + 16 * k + iota for k in range(8)]
                for s in range(8):
                    col = jnp.full((16,), jj * 8 + s, jnp.int32)
                    for k in range(8):
                        v = plsc.load_gather(gb, [rows[k], col])
                        sb[jj, bbl, s, pl.ds(16 * k, 16)] = v
                return carry

            lax.fori_loop(0, (_D // 8) * _BBL, body, 0)

        start_idx(u0, 0)
        wait_idx(u0, 0)
        start_gather(0)
        start_idx(u0 + 1, 1)

        def pair_body(g, carry):
            u = u0 + 2 * g
            wait_idx(u + 1, 1)
            start_gather(1)
            wait_gather(0)
            start_idx(u + 2, 0)

            @pl.when(g > 0)
            def _():
                wait_wb(u - 2, 0)

            shuffle(0)
            start_wb(u, 0)
            wait_idx(u + 2, 0)
            start_gather(0)
            wait_gather(1)
            start_idx(u + 3, 1)

            @pl.when(g > 0)
            def _():
                wait_wb(u - 1, 1)

            shuffle(1)
            start_wb(u + 1, 1)
            return carry

        lax.fori_loop(0, _UPW // 2 - 1, pair_body, 0)

        u = u0 + _UPW - 2
        wait_idx(u + 1, 1)
        start_gather(1)
        wait_gather(0)
        wait_wb(u - 2, 0)
        shuffle(0)
        start_wb(u, 0)
        wait_gather(1)
        wait_wb(u - 1, 1)
        shuffle(1)
        start_wb(u + 1, 1)
        wait_wb(u, 0)
        wait_wb(u + 1, 1)

    return gather_kernel


_GATHER = _build()


def kernel(tokens, table):
    idx = tokens.T.reshape(-1).astype(jnp.int32)
    z = _GATHER(idx, table)
    return z.transpose(2, 4, 0, 1, 3).reshape(_BATCH, _SEQ, _D)


# speedup vs baseline: 1.5672x; 1.5672x over previous
import functools

import jax
import jax.numpy as jnp
from jax import lax
from jax.experimental import pallas as pl
from jax.experimental.pallas import tpu as pltpu
from jax.experimental.pallas import tpu_sc as plsc

_BATCH = 4096
_SEQ = 200
_V = 1000000
_D = 32
_NC, _NS = 2, 16
_NW = _NC * _NS
_CB = 512
_BQ = _BATCH // _CB
_BBL = _CB // 128
_UNITS = _SEQ * _BQ
_UPW = _UNITS // _NW


def _build():
    mesh = plsc.VectorSubcoreMesh(core_axis_name="c", subcore_axis_name="s")

    @functools.partial(
        pl.kernel,
        mesh=mesh,
        out_type=jax.ShapeDtypeStruct((_SEQ, _D // 8, _BATCH * 8),
                                      jnp.float32),
        compiler_params=pltpu.CompilerParams(use_tc_tiling_on_sc=False,
                                             needs_layout_passes=False),
        scratch_types=[
            pltpu.VMEM((_CB,), jnp.int32),
            pltpu.VMEM((_CB,), jnp.int32),
            pltpu.VMEM((_CB, _D), jnp.float32),
            pltpu.VMEM((_CB, _D), jnp.float32),
            pltpu.VMEM((_CB * _D,), jnp.float32),
            pltpu.VMEM((_CB * _D,), jnp.float32),
            pltpu.SemaphoreType.DMA,
            pltpu.SemaphoreType.DMA,
            pltpu.SemaphoreType.DMA,
            pltpu.SemaphoreType.DMA,
            pltpu.SemaphoreType.DMA,
            pltpu.SemaphoreType.DMA,
        ],
    )
    def gather_kernel(idx_hbm, table_hbm, z_hbm, ib0, ib1, gb0, gb1, sb0, sb1,
                      is0, is1, gs0, gs1, ws0, ws1):
        wid = lax.axis_index("s") * _NC + lax.axis_index("c")
        u0 = wid * _UPW
        iota = lax.iota(jnp.int32, 16)
        diag = [(s0 + iota) & 7 for s0 in range(8)]
        dpat = [diag[s0] * 128 + iota for s0 in range(8)]

        ibufs = (ib0, ib1)
        gbufs = (gb0, gb1)
        sbufs = (sb0, sb1)
        isems = (is0, is1)
        gsems = (gs0, gs1)
        wsems = (ws0, ws1)

        def idx_src(u):
            l = u // _BQ
            bq = u % _BQ
            return idx_hbm.at[pl.ds(l * _BATCH + bq * _CB, _CB)]

        def start_idx(u, p):
            pltpu.async_copy(idx_src(u), ibufs[p], isems[p])

        def wait_idx(u, p):
            pltpu.make_async_copy(idx_src(u), ibufs[p], isems[p]).wait()

        def start_gather(p):
            pltpu.async_copy(table_hbm.at[ibufs[p]], gbufs[p], gsems[p])

        def wait_gather(p):
            pltpu.make_async_copy(table_hbm.at[ibufs[p]], gbufs[p],
                                  gsems[p]).wait()

        def wb_pairs(u, p):
            l = u // _BQ
            bq = u % _BQ
            return [(sbufs[p].at[pl.ds(jj * _CB * 8, _CB * 8)],
                     z_hbm.at[l, jj, pl.ds(bq * _CB * 8, _CB * 8)])
                    for jj in range(_D // 8)]

        def start_wb(u, p):
            for src, dst in wb_pairs(u, p):
                pltpu.async_copy(src, dst, wsems[p])

        def wait_wb(u, p):
            for src, dst in wb_pairs(u, p):
                pltpu.make_async_copy(src, dst, wsems[p]).wait()

        def shuffle(p):
            gb = gbufs[p]
            sb = sbufs[p]

            def body(t, carry):
                jj = t // _BBL
                bbl = t % _BBL
                for s0 in range(8):
                    colv = diag[s0] + jj * 8
                    for k in range(8):
                        rowv = iota + (bbl * 128 + 16 * k)
                        v = plsc.load_gather(gb, [rowv, colv])
                        dstv = dpat[s0] + (jj * 4096 + bbl * 1024 + 16 * k)
                        plsc.store_scatter(sb, [dstv], v)
                return carry

            lax.fori_loop(0, (_D // 8) * _BBL, body, 0)

        start_idx(u0, 0)
        wait_idx(u0, 0)
        start_gather(0)
        start_idx(u0 + 1, 1)

        def pair_body(g, carry):
            u = u0 + 2 * g
            wait_idx(u + 1, 1)
            start_gather(1)
            wait_gather(0)
            start_idx(u + 2, 0)

            @pl.when(g > 0)
            def _():
                wait_wb(u - 2, 0)

            shuffle(0)
            start_wb(u, 0)
            wait_idx(u + 2, 0)
            start_gather(0)
            wait_gather(1)
            start_idx(u + 3, 1)

            @pl.when(g > 0)
            def _():
                wait_wb(u - 1, 1)

            shuffle(1)
            start_wb(u + 1, 1)
            return carry

        lax.fori_loop(0, _UPW // 2 - 1, pair_body, 0)

        u = u0 + _UPW - 2
        wait_idx(u + 1, 1)
        start_gather(1)
        wait_gather(0)
        wait_wb(u - 2, 0)
        shuffle(0)
        start_wb(u, 0)
        wait_gather(1)
        wait_wb(u - 1, 1)
        shuffle(1)
        start_wb(u + 1, 1)
        wait_wb(u, 0)
        wait_wb(u + 1, 1)

    return gather_kernel


_GATHER = _build()


def kernel(tokens, table):
    idx = tokens.T.reshape(-1).astype(jnp.int32)
    z = _GATHER(idx, table)
    z = z.reshape(_SEQ, _D // 8, _BATCH // 128, 8, 128)
    return z.transpose(2, 4, 0, 1, 3).reshape(_BATCH, _SEQ, _D)
